# Initial kernel scaffold; baseline (speedup 1.0000x reference)
import jax
import jax.numpy as jnp
from jax.experimental import pallas as pl


def kernel(col_0, col_1, col_2, col_3, col_4, col_5, col_6, col_7, col_8, col_9, col_10, col_11, col_12, col_13, col_14, col_15, col_16, col_17, col_18, col_19, col_20, col_21, col_22, col_23, col_24, col_25, table_0, table_1, table_2, table_3, table_4, table_5, table_6, table_7, table_8, table_9, table_10, table_11, table_12, table_13, table_14, table_15, table_16, table_17, table_18, table_19, table_20, table_21, table_22, table_23, table_24, table_25):
    raise NotImplementedError("write your pallas kernel here")


# speedup vs baseline: 4.9007x; 4.9007x over previous
import functools

import jax
import jax.numpy as jnp
from jax import lax
from jax.experimental import pallas as pl
from jax.experimental.pallas import tpu as pltpu
from jax.experimental.pallas import tpu_sc as plsc

_NUM_COLS = 26
_VOCAB = 1000
_DIM = 128
_BATCH = 16384
_NC = 2
_NS = 16
_NW = _NC * _NS
_CHUNK = _BATCH // _NW
_SUB = 128
_NSUB = _CHUNK // _SUB


def _build():
    mesh = plsc.VectorSubcoreMesh(core_axis_name="c", subcore_axis_name="s")

    @functools.partial(
        pl.kernel,
        mesh=mesh,
        out_type=jax.ShapeDtypeStruct((_BATCH, _NUM_COLS * _DIM), jnp.float32),
        scratch_types=[
            pltpu.VMEM((_SUB,), jnp.int32),
            pltpu.VMEM((_NSUB, _SUB, _DIM), jnp.float32),
            pltpu.SemaphoreType.DMA,
        ],
    )
    def k(tbl_hbm, idx_hbm, out_hbm, idx_v, rows_v, sem):
        wid = lax.axis_index("s") * _NC + lax.axis_index("c")
        base = wid * _CHUNK

        def body(t, carry):
            for j in range(_NSUB):
                pltpu.sync_copy(idx_hbm.at[t * (_NW * _NSUB) + wid * _NSUB + j], idx_v)
                pltpu.async_copy(tbl_hbm.at[idx_v], rows_v.at[j], sem).wait()
                pltpu.sync_copy(
                    rows_v.at[j],
                    out_hbm.at[pl.ds(base + j * _SUB, _SUB), pl.ds(t * _DIM, _DIM)],
                )
            return carry

        lax.fori_loop(0, _NUM_COLS, body, 0)

    return k


_GATHER_CACHE = []


def _gather_fn():
    if not _GATHER_CACHE:
        _GATHER_CACHE.append(_build())
    return _GATHER_CACHE[0]


def kernel(col_0, col_1, col_2, col_3, col_4, col_5, col_6, col_7, col_8, col_9, col_10, col_11, col_12, col_13, col_14, col_15, col_16, col_17, col_18, col_19, col_20, col_21, col_22, col_23, col_24, col_25, table_0, table_1, table_2, table_3, table_4, table_5, table_6, table_7, table_8, table_9, table_10, table_11, table_12, table_13, table_14, table_15, table_16, table_17, table_18, table_19, table_20, table_21, table_22, table_23, table_24, table_25):
    cols = jnp.stack([
        col_0, col_1, col_2, col_3, col_4, col_5, col_6, col_7, col_8, col_9,
        col_10, col_11, col_12, col_13, col_14, col_15, col_16, col_17,
        col_18, col_19, col_20, col_21, col_22, col_23, col_24, col_25,
    ])
    offs = (jnp.arange(_NUM_COLS, dtype=jnp.int32) * _VOCAB)[:, None]
    idx = (cols + offs).reshape(_NUM_COLS * _BATCH // _SUB, _SUB)
    tbl = jnp.concatenate([
        table_0, table_1, table_2, table_3, table_4, table_5, table_6,
        table_7, table_8, table_9, table_10, table_11, table_12, table_13,
        table_14, table_15, table_16, table_17, table_18, table_19, table_20,
        table_21, table_22, table_23, table_24, table_25,
    ], axis=0)
    return _gather_fn()(tbl, idx)


# speedup vs baseline: 7.8076x; 1.5932x over previous
import functools

import jax
import jax.numpy as jnp
from jax import lax
from jax.experimental import pallas as pl
from jax.experimental.pallas import tpu as pltpu
from jax.experimental.pallas import tpu_sc as plsc

_NUM_COLS = 26
_VOCAB = 1000
_DIM = 128
_BATCH = 16384
_NC = 2
_NS = 16
_NW = _NC * _NS
_CHUNK = _BATCH // _NW
_SUB = 128
_NSUB = _CHUNK // _SUB


def _build():
    mesh = plsc.VectorSubcoreMesh(core_axis_name="c", subcore_axis_name="s")

    @functools.partial(
        pl.kernel,
        mesh=mesh,
        out_type=jax.ShapeDtypeStruct((_BATCH, _NUM_COLS * _DIM), jnp.float32),
        scratch_types=[
            pltpu.VMEM((_NUM_COLS, _NSUB, _SUB), jnp.int32),
            pltpu.VMEM((_NSUB, _SUB, _DIM), jnp.float32),
        ]
        + [pltpu.SemaphoreType.DMA] * (2 * _NSUB),
    )
    def k(tbl_hbm, idx_hbm, out_hbm, idx_v, rows_v, *sems):
        gsem = sems[:_NSUB]
        osem = sems[_NSUB:]
        wid = lax.axis_index("s") * _NC + lax.axis_index("c")
        base = wid * _CHUNK

        pltpu.sync_copy(idx_hbm.at[:, pl.ds(wid * _NSUB, _NSUB), :], idx_v)

        def gather(t, j):
            pltpu.async_copy(tbl_hbm.at[idx_v.at[t, j]], rows_v.at[j], gsem[j])

        def store(t, j):
            pltpu.async_copy(
                rows_v.at[j],
                out_hbm.at[pl.ds(base + j * _SUB, _SUB), pl.ds(t * _DIM, _DIM)],
                osem[j],
            )

        for j in range(_NSUB):
            gather(0, j)

        def body(t, carry):
            for j in range(_NSUB):
                pltpu.make_async_copy(
                    tbl_hbm.at[idx_v.at[t, j]], rows_v.at[j], gsem[j]
                ).wait()
                store(t, j)
            for j in range(_NSUB):
                pltpu.make_async_copy(
                    rows_v.at[j],
                    out_hbm.at[pl.ds(base + j * _SUB, _SUB), pl.ds(t * _DIM, _DIM)],
                    osem[j],
                ).wait()

                @pl.when(t < _NUM_COLS - 1)
                def _():
                    gather(t + 1, j)

            return carry

        lax.fori_loop(0, _NUM_COLS, body, 0)

    return k


_GATHER_CACHE = []


def _gather_fn():
    if not _GATHER_CACHE:
        _GATHER_CACHE.append(_build())
    return _GATHER_CACHE[0]


def kernel(col_0, col_1, col_2, col_3, col_4, col_5, col_6, col_7, col_8, col_9, col_10, col_11, col_12, col_13, col_14, col_15, col_16, col_17, col_18, col_19, col_20, col_21, col_22, col_23, col_24, col_25, table_0, table_1, table_2, table_3, table_4, table_5, table_6, table_7, table_8, table_9, table_10, table_11, table_12, table_13, table_14, table_15, table_16, table_17, table_18, table_19, table_20, table_21, table_22, table_23, table_24, table_25):
    cols = jnp.stack([
        col_0, col_1, col_2, col_3, col_4, col_5, col_6, col_7, col_8, col_9,
        col_10, col_11, col_12, col_13, col_14, col_15, col_16, col_17,
        col_18, col_19, col_20, col_21, col_22, col_23, col_24, col_25,
    ])
    offs = (jnp.arange(_NUM_COLS, dtype=jnp.int32) * _VOCAB)[:, None]
    idx = (cols + offs).reshape(_NUM_COLS, _NW * _NSUB, _SUB)
    tbl = jnp.concatenate([
        table_0, table_1, table_2, table_3, table_4, table_5, table_6,
        table_7, table_8, table_9, table_10, table_11, table_12, table_13,
        table_14, table_15, table_16, table_17, table_18, table_19, table_20,
        table_21, table_22, table_23, table_24, table_25,
    ], axis=0)
    return _gather_fn()(tbl, idx)
